# baseline (speedup 1.0000x reference)
import functools

import jax
import jax.numpy as jnp
from jax import lax
from jax.experimental import pallas as pl
from jax.experimental.pallas import tpu as pltpu
from jax.experimental.pallas import tpu_sc as plsc

L = 16
NC = 2
NS = 16
NW = NC * NS

B, SEQ = 16384, 200
COLS_W = B // NW
RCHUNK = 40
NCHUNK = SEQ // RCHUNK
VROW = COLS_W // L


@functools.partial(
    pl.kernel,
    out_type=jax.ShapeDtypeStruct((NW, L), jnp.float32),
    mesh=plsc.VectorSubcoreMesh(core_axis_name="c", subcore_axis_name="s"),
    compiler_params=pltpu.CompilerParams(needs_layout_passes=False),
    scratch_types=[
        pltpu.VMEM((RCHUNK, COLS_W), jnp.int32),
        pltpu.VMEM((RCHUNK, COLS_W), jnp.int32),
        pltpu.VMEM((L, L), jnp.float32),
        pltpu.VMEM((L, L), jnp.float32),
        pltpu.VMEM((1, L), jnp.float32),
        pltpu.VMEM((10 * L,), jnp.int32),
        pltpu.SemaphoreType.DMA,
        pltpu.SemaphoreType.DMA,
    ],
)
def _sc_sum(xt_hbm, ea_hbm, eb_hbm, out_hbm,
            buf0, buf1, tab_a, tab_b, acc_ref, hist, sem0, sem1):
    cid = lax.axis_index("c")
    sid = lax.axis_index("s")
    wid = sid * NC + cid
    col0 = wid * COLS_W

    pltpu.sync_copy(ea_hbm, tab_a)
    pltpu.sync_copy(eb_hbm, tab_b)
    s = jnp.zeros((L,), jnp.float32)
    for k in range(L):
        s = s + tab_a[k] + tab_b[k]

    izeros = jnp.zeros((L,), jnp.int32)
    for v in range(10):
        hist[pl.ds(v * L, L)] = izeros
    lane = lax.iota(jnp.int32, L)
    ones = jnp.ones((L,), jnp.int32)

    bufs = (buf0, buf1)
    sems = (sem0, sem1)

    def dma(c, buf, sem):
        return pltpu.make_async_copy(
            xt_hbm.at[pl.ds(c * RCHUNK, RCHUNK), pl.ds(col0, COLS_W)],
            buf, sem)

    dma(0, buf0, sem0).start()
    for c in range(NCHUNK):
        buf, sem = bufs[c % 2], sems[c % 2]
        if c + 1 < NCHUNK:
            dma(c + 1, bufs[(c + 1) % 2], sems[(c + 1) % 2]).start()
        dma(c, buf, sem).wait()

        def body(r, a, buf=buf):
            for j in range(VROW):
                idx = buf[r, pl.ds(j * L, L)]
                plsc.addupdate_scatter(hist, [idx * L + lane], ones)
            return a

        lax.fori_loop(0, RCHUNK, body, 0)

    acc = jnp.zeros((L,), jnp.float32)
    for v in range(10):
        acc = acc + s[v] * hist[pl.ds(v * L, L)].astype(jnp.float32)

    acc_ref[0, :] = acc
    pltpu.sync_copy(acc_ref, out_hbm.at[pl.ds(wid, 1)])


def kernel(x, emb, emb2):
    xt = x.astype(jnp.int32).T
    ea = jnp.zeros((L, L), jnp.float32).at[:10, :10].set(emb.T)
    eb = jnp.zeros((L, L), jnp.float32).at[:10, :10].set(emb2.T)
    partials = _sc_sum(xt, ea, eb)
    return jnp.sum(partials)


# speedup vs baseline: 2.0075x; 2.0075x over previous
import functools

import jax
import jax.numpy as jnp
from jax import lax
from jax.experimental import pallas as pl
from jax.experimental.pallas import tpu as pltpu
from jax.experimental.pallas import tpu_sc as plsc

L = 16
NC = 2
NS = 16
NW = NC * NS

B, SEQ = 16384, 200
COLS_W = B // NW
RCHUNK = 40
NCHUNK = SEQ // RCHUNK
VROW = COLS_W // L


@functools.partial(
    pl.kernel,
    out_type=jax.ShapeDtypeStruct((NW, L), jnp.float32),
    mesh=plsc.VectorSubcoreMesh(core_axis_name="c", subcore_axis_name="s"),
    compiler_params=pltpu.CompilerParams(needs_layout_passes=False),
    scratch_types=[
        pltpu.VMEM((RCHUNK, COLS_W), jnp.int32),
        pltpu.VMEM((RCHUNK, COLS_W), jnp.int32),
        pltpu.VMEM((L, L), jnp.float32),
        pltpu.VMEM((L, L), jnp.float32),
        pltpu.VMEM((L,), jnp.float32),
        pltpu.VMEM((1, L), jnp.float32),
        pltpu.SemaphoreType.DMA,
        pltpu.SemaphoreType.DMA,
    ],
)
def _sc_sum(xt_hbm, ea_hbm, eb_hbm, out_hbm,
            buf0, buf1, tab_a, tab_b, s_ref, acc_ref, sem0, sem1):
    cid = lax.axis_index("c")
    sid = lax.axis_index("s")
    wid = sid * NC + cid
    col0 = wid * COLS_W

    pltpu.sync_copy(ea_hbm, tab_a)
    pltpu.sync_copy(eb_hbm, tab_b)
    s = jnp.zeros((L,), jnp.float32)
    for k in range(L):
        s = s + tab_a[k] + tab_b[k]
    s_ref[...] = s

    bufs = (buf0, buf1)
    sems = (sem0, sem1)

    def dma(c, buf, sem):
        return pltpu.make_async_copy(
            xt_hbm.at[pl.ds(c * RCHUNK, RCHUNK), pl.ds(col0, COLS_W)],
            buf, sem)

    dma(0, buf0, sem0).start()
    acc = jnp.zeros((L,), jnp.float32)
    for c in range(NCHUNK):
        buf, sem = bufs[c % 2], sems[c % 2]
        if c + 1 < NCHUNK:
            dma(c + 1, bufs[(c + 1) % 2], sems[(c + 1) % 2]).start()
        dma(c, buf, sem).wait()

        def body(r, a, buf=buf):
            for j in range(VROW):
                idx = buf[r, pl.ds(j * L, L)]
                a = a + plsc.load_gather(s_ref, [idx])
            return a

        acc = lax.fori_loop(0, RCHUNK, body, acc)

    acc_ref[0, :] = acc
    pltpu.sync_copy(acc_ref, out_hbm.at[pl.ds(wid, 1)])


def kernel(x, emb, emb2):
    xt = x.astype(jnp.int32).T
    ea = jnp.zeros((L, L), jnp.float32).at[:10, :10].set(emb.T)
    eb = jnp.zeros((L, L), jnp.float32).at[:10, :10].set(emb2.T)
    partials = _sc_sum(xt, ea, eb)
    return jnp.sum(partials)


# speedup vs baseline: 2.0833x; 1.0378x over previous
import functools

import jax
import jax.numpy as jnp
from jax import lax
from jax.experimental import pallas as pl
from jax.experimental.pallas import tpu as pltpu
from jax.experimental.pallas import tpu_sc as plsc

L = 16
NC = 2
NS = 16
NW = NC * NS
V = 10

B, SEQ = 16384, 200
COLS_W = B // NW
RCHUNK = 40
NCHUNK = SEQ // RCHUNK
VROW = COLS_W // L
PAIRS = VROW // 2


@functools.partial(
    pl.kernel,
    out_type=jax.ShapeDtypeStruct((NW, L), jnp.float32),
    mesh=plsc.VectorSubcoreMesh(core_axis_name="c", subcore_axis_name="s"),
    compiler_params=pltpu.CompilerParams(needs_layout_passes=False),
    scratch_types=[
        pltpu.VMEM((RCHUNK, COLS_W), jnp.int32),
        pltpu.VMEM((RCHUNK, COLS_W), jnp.int32),
        pltpu.VMEM((V, V), jnp.float32),
        pltpu.VMEM((V, V), jnp.float32),
        pltpu.VMEM((L * L,), jnp.float32),
        pltpu.VMEM((1, L), jnp.float32),
        pltpu.SemaphoreType.DMA,
        pltpu.SemaphoreType.DMA,
    ],
)
def _sc_sum(xt_hbm, ea_hbm, eb_hbm, out_hbm,
            buf0, buf1, tab_a, tab_b, pair, acc_ref, sem0, sem1):
    cid = lax.axis_index("c")
    sid = lax.axis_index("s")
    wid = sid * NC + cid
    col0 = wid * COLS_W

    pltpu.sync_copy(ea_hbm, tab_a)
    pltpu.sync_copy(eb_hbm, tab_b)
    rows = lax.iota(jnp.int32, L)
    keep = rows < V
    zeros = jnp.zeros((L,), jnp.float32)
    s = zeros
    for k in range(V):
        col = jnp.full((L,), k, jnp.int32)
        s = s + plsc.load_gather(tab_a, [rows, col], mask=keep)
        s = s + plsc.load_gather(tab_b, [rows, col], mask=keep)
    s = jnp.where(keep, s, zeros)

    for a in range(V):
        pair[pl.ds(a * L, L)] = s[a] + s

    bufs = (buf0, buf1)
    sems = (sem0, sem1)

    def dma(c, buf, sem):
        return pltpu.make_async_copy(
            xt_hbm.at[pl.ds(c * RCHUNK, RCHUNK), pl.ds(col0, COLS_W)],
            buf, sem)

    dma(0, buf0, sem0).start()
    acc = zeros
    for c in range(NCHUNK):
        buf, sem = bufs[c % 2], sems[c % 2]
        if c + 1 < NCHUNK:
            dma(c + 1, bufs[(c + 1) % 2], sems[(c + 1) % 2]).start()
        dma(c, buf, sem).wait()

        def body(r, a, buf=buf):
            for j in range(PAIRS):
                ia = buf[r, pl.ds(2 * j * L, L)]
                ib = buf[r, pl.ds((2 * j + 1) * L, L)]
                a = a + plsc.load_gather(pair, [ia * L + ib])
            return a

        acc = lax.fori_loop(0, RCHUNK, body, acc)

    acc_ref[0, :] = acc
    pltpu.sync_copy(acc_ref, out_hbm.at[pl.ds(wid, 1)])


def kernel(x, emb, emb2):
    xt = x.astype(jnp.int32).T
    partials = _sc_sum(xt, emb, emb2)
    return jnp.sum(partials)
